# Initial kernel scaffold; baseline (speedup 1.0000x reference)
import jax
import jax.numpy as jnp
from jax.experimental import pallas as pl


def kernel(x, s1_conv_w, s1_conv_b, s1_bn_g, s1_bn_b, s1_fc_w, s1_fc_b, s2_conv_w, s2_conv_b, s2_bn_g, s2_bn_b, s2_fc_w, s2_fc_b):
    raise NotImplementedError("write your pallas kernel here")


# speedup vs baseline: 1.6909x; 1.6909x over previous
import numpy as np
import jax
import jax.numpy as jnp
from jax.experimental import pallas as pl

_B_BLOCK = 256
_SQRT1PEPS = np.float32(np.sqrt(np.float32(1.0 + 1e-5)))
_RECIP9 = np.float32(1.0) / np.float32(9.0)


def _build_t1():
    t1 = np.zeros((4, 81, 9, 9), dtype=np.float32)
    for t, (du, dv) in enumerate([(0, 0), (0, 1), (1, 0), (1, 1)]):
        for i in range(3):
            for j in range(3):
                for u in range(3):
                    for v in range(3):
                        rs = (2 * i + du + u) * 9 + (2 * j + dv + v)
                        t1[t, rs, u * 3 + v, i * 3 + j] = 1.0
    return t1


def _build_t2():
    t2 = np.zeros((4, 81, 25, 4), dtype=np.float32)
    for t, (du, dv) in enumerate([(0, 0), (0, 1), (1, 0), (1, 1)]):
        for i in range(2):
            for j in range(2):
                for u in range(5):
                    for v in range(5):
                        rs = (2 * i + du + u) * 9 + (2 * j + dv + v)
                        t2[t, rs, u * 5 + v, i * 2 + j] = 1.0
    return t2


_T1 = _build_t1()
_T2 = _build_t2()
_EYE45 = np.eye(45, dtype=np.float32)


def _fused_kernel(xp_ref, w1_ref, cb1_ref, g1_ref, bb1_ref, fw1_ref, fb1_ref,
                  w2_ref, b2_ref, g_ref, b90_ref,
                  s1_out_ref, out_ref, idx_ref):
    f32 = jnp.float32
    xp = xp_ref[...]

    cs0 = (xp[:, 0, :] + xp[:, 3, :]) + xp[:, 6, :]
    cs1 = (xp[:, 1, :] + xp[:, 4, :]) + xp[:, 7, :]
    cs2 = (xp[:, 2, :] + xp[:, 5, :]) + xp[:, 8, :]
    pooled = ((cs0 + cs1) + cs2) * _RECIP9
    pb = pooled.astype(jnp.bfloat16)

    m1 = None
    for t in range(4):
        c = jnp.dot(pb, w1_ref[t], preferred_element_type=f32)
        c = ((c + cb1_ref[...]) / _SQRT1PEPS) * g1_ref[...] + bb1_ref[...]
        m1 = c if m1 is None else jnp.maximum(m1, c)
    h1 = jnp.maximum(m1, jnp.float32(0.0))
    s1 = jnp.dot(h1.astype(jnp.bfloat16), fw1_ref[...],
                 preferred_element_type=f32) + fb1_ref[...]
    s1_out_ref[...] = s1

    bb = s1.shape[0]
    iota10 = jax.lax.broadcasted_iota(jnp.int32, (bb, 10), 1)
    big = jnp.int32(127)
    vmax = jnp.max(s1, axis=1, keepdims=True)
    idx_a = jnp.min(jnp.where(s1 >= vmax, iota10, big), axis=1, keepdims=True)
    s1m = jnp.where(iota10 == idx_a, jnp.float32(-1e30), s1)
    vmax2 = jnp.max(s1m, axis=1, keepdims=True)
    idx_b = jnp.min(jnp.where(s1m >= vmax2, iota10, big), axis=1, keepdims=True)
    i0 = jnp.minimum(idx_a, idx_b)
    i1 = jnp.maximum(idx_a, idx_b)
    idx_ref[:, 0:1] = i0
    idx_ref[:, 1:2] = i1
    expert = (i0 * (19 - i0)) // 2 + (i1 - i0 - 1)

    m2 = None
    for t in range(4):
        c = jnp.dot(pb, w2_ref[t], preferred_element_type=f32)
        m2 = c if m2 is None else jnp.maximum(m2, c)
    h2 = jax.nn.relu(m2 + b2_ref[...])
    out2 = jnp.dot(h2.astype(jnp.bfloat16), g_ref[...],
                   preferred_element_type=f32) + b90_ref[...]

    iota90 = jax.lax.broadcasted_iota(jnp.int32, (bb, 90), 1)
    zero = jnp.float32(0.0)
    s20 = jnp.sum(jnp.where(iota90 == 2 * expert, out2, zero), axis=1,
                  keepdims=True)
    s21 = jnp.sum(jnp.where(iota90 == 2 * expert + 1, out2, zero), axis=1,
                  keepdims=True)

    o = jnp.where(iota10 == i0, s20,
                  jnp.where(iota10 == i1, s21, jnp.float32(-100.0)))
    m = jnp.max(o, axis=1, keepdims=True)
    osh = o - m
    lse = jnp.log(jnp.sum(jnp.exp(osh), axis=1, keepdims=True))
    out_ref[...] = osh - lse


def kernel(x, s1_conv_w, s1_conv_b, s1_bn_g, s1_bn_b, s1_fc_w, s1_fc_b,
           s2_conv_w, s2_conv_b, s2_bn_g, s2_bn_b, s2_fc_w, s2_fc_b):
    B = x.shape[0]
    xpatch = x.reshape(B, 28, 28)[:, :27, :27].reshape(
        B, 9, 3, 9, 3).transpose(0, 2, 4, 1, 3).reshape(B, 9, 81)

    w1s = s1_conv_w.reshape(20, 9)
    w1stack = jnp.einsum('trup,cu->trcp', _T1, w1s).reshape(
        4, 81, 180).astype(jnp.bfloat16)
    cb1 = jnp.repeat(s1_conv_b, 9).reshape(1, 180)
    g1 = jnp.repeat(s1_bn_g, 9).reshape(1, 180)
    bb1 = jnp.repeat(s1_bn_b, 9).reshape(1, 180)
    fw1t = s1_fc_w.T.astype(jnp.bfloat16)
    fb1 = s1_fc_b.reshape(1, 10)

    inv = 1.0 / np.sqrt(1.0 + 1e-5)
    scale2 = s2_bn_g * inv
    w2s = s2_conv_w.reshape(45, 15, 25) * scale2[..., None]
    w2stack = jnp.einsum('trup,ecu->trecp', _T2, w2s).reshape(
        4, 81, 2700).astype(jnp.bfloat16)
    b2eff = jnp.repeat((s2_conv_b * scale2 + s2_bn_b).reshape(675),
                       4).reshape(1, 2700)
    fw2t = s2_fc_w.transpose(0, 2, 1)
    g = jnp.einsum('ef,edo->edfo', _EYE45, fw2t).reshape(
        2700, 90).astype(jnp.bfloat16)
    b90 = s2_fc_b.reshape(1, 90)

    bb = _B_BLOCK
    grid = (B // bb,)
    full = lambda shape: pl.BlockSpec(shape, lambda i: (0,) * len(shape))
    s1_out, out, idx = pl.pallas_call(
        _fused_kernel,
        grid=grid,
        in_specs=[
            pl.BlockSpec((bb, 9, 81), lambda i: (i, 0, 0)),
            full((4, 81, 180)),
            full((1, 180)),
            full((1, 180)),
            full((1, 180)),
            full((180, 10)),
            full((1, 10)),
            full((4, 81, 2700)),
            full((1, 2700)),
            full((2700, 90)),
            full((1, 90)),
        ],
        out_specs=[
            pl.BlockSpec((bb, 10), lambda i: (i, 0)),
            pl.BlockSpec((bb, 10), lambda i: (i, 0)),
            pl.BlockSpec((bb, 2), lambda i: (i, 0)),
        ],
        out_shape=[
            jax.ShapeDtypeStruct((B, 10), jnp.float32),
            jax.ShapeDtypeStruct((B, 10), jnp.float32),
            jax.ShapeDtypeStruct((B, 2), jnp.int32),
        ],
    )(xpatch, w1stack, cb1, g1, bb1, fw1t, fb1, w2stack, b2eff, g, b90)
    return (s1_out, out, idx)
